# baseline (speedup 1.0000x reference)
import functools

import jax
import jax.numpy as jnp
from jax import lax
from jax.experimental import pallas as pl
from jax.experimental.pallas import tpu as pltpu
from jax.experimental.pallas import tpu_sc as plsc

USER = 4000
ITEM = 6000
N = USER + ITEM
D = 128
HALF = 64
NLAYER = 3
E = 320000

NS = 16
B = 400
NSET = 2
EPT = E // NS
NBATCH = EPT // B
NP = 10240
ROWS_PT = NP // NS
CH = 128
NCH = ROWS_PT // CH
QL = HALF // 16

_mesh = plsc.VectorSubcoreMesh(core_axis_name="c", subcore_axis_name="s")


def _bcast_lane(vec, lane):
    idx = jnp.full((16, 1), lane, jnp.int32)
    dnums = lax.GatherDimensionNumbers(
        offset_dims=(), collapsed_slice_dims=(0,), start_index_map=(0,))
    return lax.gather(vec, idx, dnums, slice_sizes=(1,),
                      mode=lax.GatherScatterMode.PROMISE_IN_BOUNDS)


@functools.partial(
    pl.kernel,
    out_type=jax.ShapeDtypeStruct((2 * NP, HALF), jnp.float32),
    mesh=_mesh,
    compiler_params=pltpu.CompilerParams(use_tc_tiling_on_sc=False),
    scratch_types=[
        pltpu.VMEM_SHARED((NP, HALF), jnp.float32),
        *([pltpu.VMEM((2, B), jnp.int32)] * NSET),
        *([pltpu.VMEM((B,), jnp.int32)] * NSET),
        *([pltpu.VMEM((B, HALF), jnp.float32)] * NSET),
        pltpu.VMEM((CH, HALF), jnp.float32),
        pltpu.VMEM((CH, HALF), jnp.float32),
        pltpu.VMEM((CH, HALF), jnp.float32),
        pltpu.SemaphoreType.DMA,
        pltpu.SemaphoreType.DMA,
        pltpu.SemaphoreType.DMA,
    ],
)
def _lightgcn(x0, esw, edst, out, acc,
              e0b, e1b, d0b, d1b, r0b, r1b,
              cb0, cb1, zbuf, gsem, ssem, esem):
    c = lax.axis_index("c")
    s = lax.axis_index("s")
    row0 = s * ROWS_PT
    cN = c * NP
    myrow = pl.multiple_of(cN + row0, 8)
    accrow = pl.multiple_of(row0, 8)
    zero16 = jnp.zeros((16,), jnp.float32)
    sets = ((e0b, d0b, r0b), (e1b, d1b, r1b))

    @functools.partial(plsc.parallel_loop(0, CH, 1, unroll=4))
    def zrow(r):
        for q in range(QL):
            zbuf[r, pl.ds(q * 16, 16)] = zero16

    for k in range(NCH):
        pltpu.sync_copy(zbuf, acc.at[pl.ds(accrow + k * CH, CH)])

    for k in range(NCH):
        pltpu.sync_copy(x0.at[pl.ds(myrow + k * CH, CH)], cb0)
        pltpu.sync_copy(cb0, out.at[pl.ds(myrow + k * CH, CH)])
    plsc.subcore_barrier()

    def load_and_fire(p, b, drain):
        ebuf, dstb, rows_ = sets[p]
        e0 = s * EPT + b * B
        eh = pltpu.async_copy(esw.at[:, pl.ds(e0, B)], ebuf, esem)
        if drain:
            pltpu.make_async_copy(rows_, acc.at[dstb], ssem).wait()
        pltpu.sync_copy(edst.at[pl.ds(e0, B)], dstb)
        eh.wait()

        @functools.partial(plsc.parallel_loop(0, B // 16, 1, unroll=2))
        def adj(i):
            sl = pl.ds(i * 16, 16)
            ebuf[0, sl] = ebuf[0, sl] + cN
        return pltpu.async_copy(out.at[ebuf.at[0]], rows_, gsem)

    def finish(p):
        ebuf, dstb, rows_ = sets[p]
        pltpu.make_async_copy(out.at[ebuf.at[0]], rows_, gsem).wait()

        @functools.partial(plsc.parallel_loop(0, B // 16, 1, unroll=4))
        def scale_group(g):
            wv = lax.bitcast_convert_type(ebuf[1, pl.ds(g * 16, 16)],
                                          jnp.float32)
            for l in range(16):
                wbc = _bcast_lane(wv, l)
                e = g * 16 + l
                for q in range(QL):
                    sl = pl.ds(q * 16, 16)
                    rows_[e, sl] = rows_[e, sl] * wbc
        return pltpu.async_copy(rows_, acc.at[dstb], ssem, add=True)

    def layer_body(_, carry):
        load_and_fire(0, 0, drain=False)
        load_and_fire(1, 1, drain=False)
        finish(0)
        load_and_fire(0, 2, drain=True)
        finish(1)

        def ring_body(j, cr):
            load_and_fire(1, 2 * j + 1, drain=True)
            finish(0)
            load_and_fire(0, 2 * j + 2, drain=True)
            finish(1)
            return cr
        lax.fori_loop(1, NBATCH // 2 - 1, ring_body, 0)
        load_and_fire(1, NBATCH - 1, drain=True)
        finish(0)
        finish(1)
        for p in range(NSET):
            ebuf, dstb, rows_ = sets[p]
            pltpu.make_async_copy(rows_, acc.at[dstb], ssem).wait()
        plsc.subcore_barrier()

        for k in range(NCH):
            pltpu.sync_copy(x0.at[pl.ds(myrow + k * CH, CH)], cb0)
            pltpu.sync_copy(acc.at[pl.ds(accrow + k * CH, CH)], cb1)

            @functools.partial(plsc.parallel_loop(0, CH, 1, unroll=4))
            def addrow(r):
                for q in range(QL):
                    sl = pl.ds(q * 16, 16)
                    cb0[r, sl] = cb0[r, sl] + cb1[r, sl]
            pltpu.sync_copy(cb0, out.at[pl.ds(myrow + k * CH, CH)])
            pltpu.sync_copy(zbuf, acc.at[pl.ds(accrow + k * CH, CH)])
        plsc.subcore_barrier()
        return carry

    lax.fori_loop(0, NLAYER, layer_body, 0)


def kernel(edge_index, edge_weight, uEmbeds, iEmbeds):
    emb = jnp.concatenate([uEmbeds, iEmbeds], axis=0)
    pad = jnp.zeros((NP - N, HALF), jnp.float32)
    x0 = jnp.concatenate([emb[:, :HALF], pad, emb[:, HALF:], pad], axis=0)
    esw = jnp.stack([edge_index[1],
                     lax.bitcast_convert_type(edge_weight, jnp.int32)])
    out2 = _lightgcn(x0, esw, edge_index[0])
    out = jnp.concatenate([out2[:N], out2[NP:NP + N]], axis=1)
    return (out[:USER], out[USER:])


# speedup vs baseline: 1.0836x; 1.0836x over previous
import functools

import jax
import jax.numpy as jnp
from jax import lax
from jax.experimental import pallas as pl
from jax.experimental.pallas import tpu as pltpu
from jax.experimental.pallas import tpu_sc as plsc

USER = 4000
ITEM = 6000
N = USER + ITEM
D = 128
HALF = 64
NLAYER = 3
E = 320000

NS = 16
B = 400
NSET = 2
EPT = E // NS
NBATCH = EPT // B
NP = 10240
ROWS_PT = NP // NS
CH = 64
NCH = ROWS_PT // CH
QL = HALF // 16

_mesh = plsc.VectorSubcoreMesh(core_axis_name="c", subcore_axis_name="s")


def _bcast_lane(vec, lane):
    idx = jnp.full((16, 1), lane, jnp.int32)
    dnums = lax.GatherDimensionNumbers(
        offset_dims=(), collapsed_slice_dims=(0,), start_index_map=(0,))
    return lax.gather(vec, idx, dnums, slice_sizes=(1,),
                      mode=lax.GatherScatterMode.PROMISE_IN_BOUNDS)


@functools.partial(
    pl.kernel,
    out_type=jax.ShapeDtypeStruct((2 * NP, HALF), jnp.float32),
    mesh=_mesh,
    compiler_params=pltpu.CompilerParams(use_tc_tiling_on_sc=False),
    scratch_types=[
        pltpu.VMEM_SHARED((NP, HALF), jnp.float32),
        *([pltpu.VMEM((2, B), jnp.int32)] * NSET),
        pltpu.VMEM((NBATCH, B), jnp.int32),
        *([pltpu.VMEM((B, HALF), jnp.float32)] * NSET),
        pltpu.VMEM((CH, HALF), jnp.float32),
        pltpu.VMEM((CH, HALF), jnp.float32),
        pltpu.VMEM((CH, HALF), jnp.float32),
        pltpu.SemaphoreType.DMA,
        pltpu.SemaphoreType.DMA,
        pltpu.SemaphoreType.DMA,
    ],
)
def _lightgcn(x0, esw, edst4, out, acc,
              e0b, e1b, dstall, r0b, r1b,
              cb0, cb1, zbuf, gsem, ssem, esem):
    c = lax.axis_index("c")
    s = lax.axis_index("s")
    row0 = s * ROWS_PT
    cN = c * NP
    myrow = pl.multiple_of(cN + row0, 8)
    accrow = pl.multiple_of(row0, 8)
    zero16 = jnp.zeros((16,), jnp.float32)
    sets = ((e0b, r0b), (e1b, r1b))

    @functools.partial(plsc.parallel_loop(0, CH, 1, unroll=4))
    def zrow(r):
        for q in range(QL):
            zbuf[r, pl.ds(q * 16, 16)] = zero16

    for k in range(NCH):
        pltpu.sync_copy(zbuf, acc.at[pl.ds(accrow + k * CH, CH)])

    pltpu.sync_copy(edst4.at[s], dstall)

    for k in range(NCH):
        pltpu.sync_copy(x0.at[pl.ds(myrow + k * CH, CH)], cb0)
        pltpu.sync_copy(cb0, out.at[pl.ds(myrow + k * CH, CH)])
    plsc.subcore_barrier()

    def load_and_fire(p, b, drain):
        ebuf, rows_ = sets[p]
        e0 = s * EPT + b * B
        eh = pltpu.async_copy(esw.at[:, pl.ds(e0, B)], ebuf, esem)
        if drain:
            pltpu.make_async_copy(rows_, acc.at[dstall.at[b - NSET]],
                                  ssem).wait()
        eh.wait()

        @functools.partial(plsc.parallel_loop(0, B // 16, 1, unroll=2))
        def adj(i):
            sl = pl.ds(i * 16, 16)
            ebuf[0, sl] = ebuf[0, sl] + cN
        return pltpu.async_copy(out.at[ebuf.at[0]], rows_, gsem)

    def finish(p, b):
        ebuf, rows_ = sets[p]
        pltpu.make_async_copy(out.at[ebuf.at[0]], rows_, gsem).wait()

        @functools.partial(plsc.parallel_loop(0, B // 16, 1, unroll=4))
        def scale_group(g):
            wv = lax.bitcast_convert_type(ebuf[1, pl.ds(g * 16, 16)],
                                          jnp.float32)
            for l in range(16):
                wbc = _bcast_lane(wv, l)
                e = g * 16 + l
                for q in range(QL):
                    sl = pl.ds(q * 16, 16)
                    rows_[e, sl] = rows_[e, sl] * wbc
        return pltpu.async_copy(rows_, acc.at[dstall.at[b]], ssem, add=True)

    def layer_body(_, carry):
        load_and_fire(0, 0, drain=False)
        load_and_fire(1, 1, drain=False)
        finish(0, 0)
        load_and_fire(0, 2, drain=True)
        finish(1, 1)

        def ring_body(j, cr):
            load_and_fire(1, 2 * j + 1, drain=True)
            finish(0, 2 * j)
            load_and_fire(0, 2 * j + 2, drain=True)
            finish(1, 2 * j + 1)
            return cr
        lax.fori_loop(1, NBATCH // 2 - 1, ring_body, 0)
        load_and_fire(1, NBATCH - 1, drain=True)
        finish(0, NBATCH - 2)
        finish(1, NBATCH - 1)
        for p in range(NSET):
            ebuf, rows_ = sets[p]
            pltpu.make_async_copy(rows_, acc.at[dstall.at[NBATCH - NSET + p]],
                                  ssem).wait()
        plsc.subcore_barrier()

        for k in range(NCH):
            pltpu.sync_copy(x0.at[pl.ds(myrow + k * CH, CH)], cb0)
            pltpu.sync_copy(acc.at[pl.ds(accrow + k * CH, CH)], cb1)

            @functools.partial(plsc.parallel_loop(0, CH, 1, unroll=4))
            def addrow(r):
                for q in range(QL):
                    sl = pl.ds(q * 16, 16)
                    cb0[r, sl] = cb0[r, sl] + cb1[r, sl]
            pltpu.sync_copy(cb0, out.at[pl.ds(myrow + k * CH, CH)])
            pltpu.sync_copy(zbuf, acc.at[pl.ds(accrow + k * CH, CH)])
        plsc.subcore_barrier()
        return carry

    lax.fori_loop(0, NLAYER, layer_body, 0)


def kernel(edge_index, edge_weight, uEmbeds, iEmbeds):
    emb = jnp.concatenate([uEmbeds, iEmbeds], axis=0)
    pad = jnp.zeros((NP - N, HALF), jnp.float32)
    x0 = jnp.concatenate([emb[:, :HALF], pad, emb[:, HALF:], pad], axis=0)
    esw = jnp.stack([edge_index[1],
                     lax.bitcast_convert_type(edge_weight, jnp.int32)])
    edst4 = edge_index[0].reshape(NS, NBATCH, B)
    out2 = _lightgcn(x0, esw, edst4)
    out = jnp.concatenate([out2[:N], out2[NP:NP + N]], axis=1)
    return (out[:USER], out[USER:])


# speedup vs baseline: 1.0909x; 1.0067x over previous
import functools

import jax
import jax.numpy as jnp
from jax import lax
from jax.experimental import pallas as pl
from jax.experimental.pallas import tpu as pltpu
from jax.experimental.pallas import tpu_sc as plsc

USER = 4000
ITEM = 6000
N = USER + ITEM
D = 128
HALF = 64
NLAYER = 3
E = 320000

NS = 16
B = 400
NSET = 2
EPT = E // NS
NBATCH = EPT // B
NP = 10240
ROWS_PT = NP // NS
CH = 64
NCH = ROWS_PT // CH
QL = HALF // 16

_mesh = plsc.VectorSubcoreMesh(core_axis_name="c", subcore_axis_name="s")


def _bcast_lane(vec, lane):
    idx = jnp.full((16, 1), lane, jnp.int32)
    dnums = lax.GatherDimensionNumbers(
        offset_dims=(), collapsed_slice_dims=(0,), start_index_map=(0,))
    return lax.gather(vec, idx, dnums, slice_sizes=(1,),
                      mode=lax.GatherScatterMode.PROMISE_IN_BOUNDS)


@functools.partial(
    pl.kernel,
    out_type=jax.ShapeDtypeStruct((2 * NP, HALF), jnp.float32),
    mesh=_mesh,
    compiler_params=pltpu.CompilerParams(use_tc_tiling_on_sc=False),
    scratch_types=[
        pltpu.VMEM_SHARED((NP, HALF), jnp.float32),
        *([pltpu.VMEM((2, B), jnp.int32)] * NSET),
        pltpu.VMEM((NBATCH, B), jnp.int32),
        *([pltpu.VMEM((B, HALF), jnp.float32)] * NSET),
        pltpu.VMEM((CH, HALF), jnp.float32),
        pltpu.VMEM((CH, HALF), jnp.float32),
        pltpu.VMEM((CH, HALF), jnp.float32),
        pltpu.SemaphoreType.DMA,
        pltpu.SemaphoreType.DMA,
        pltpu.SemaphoreType.DMA,
    ],
)
def _lightgcn(x0, esw, edst4, out, acc,
              e0b, e1b, dstall, r0b, r1b,
              cb0, cb1, zbuf, gsem, ssem, esem):
    c = lax.axis_index("c")
    s = lax.axis_index("s")
    row0 = s * ROWS_PT
    cN = c * NP
    myrow = pl.multiple_of(cN + row0, 8)
    accrow = pl.multiple_of(row0, 8)
    zero16 = jnp.zeros((16,), jnp.float32)
    sets = ((e0b, r0b), (e1b, r1b))

    @functools.partial(plsc.parallel_loop(0, CH, 1, unroll=4))
    def zrow(r):
        for q in range(QL):
            zbuf[r, pl.ds(q * 16, 16)] = zero16

    for k in range(NCH):
        pltpu.sync_copy(zbuf, acc.at[pl.ds(accrow + k * CH, CH)])

    pltpu.sync_copy(edst4.at[s], dstall)

    for k in range(NCH):
        pltpu.sync_copy(x0.at[pl.ds(myrow + k * CH, CH)], cb0)
        pltpu.sync_copy(cb0, out.at[pl.ds(myrow + k * CH, CH)])
    plsc.subcore_barrier()

    def fire_eload(p, b):
        ebuf, rows_ = sets[p]
        e0 = s * EPT + b * B
        pltpu.async_copy(esw.at[:, pl.ds(e0, B)], ebuf, esem)

    def load_and_fire(p, b, drain):
        ebuf, rows_ = sets[p]
        e0 = s * EPT + b * B
        if drain:
            pltpu.make_async_copy(rows_, acc.at[dstall.at[b - NSET]],
                                  ssem).wait()
        pltpu.make_async_copy(esw.at[:, pl.ds(e0, B)], ebuf, esem).wait()

        @functools.partial(plsc.parallel_loop(0, B // 16, 1, unroll=2))
        def adj(i):
            sl = pl.ds(i * 16, 16)
            ebuf[0, sl] = ebuf[0, sl] + cN
        return pltpu.async_copy(out.at[ebuf.at[0]], rows_, gsem)

    def finish(p, b, fire_next=True):
        ebuf, rows_ = sets[p]
        pltpu.make_async_copy(out.at[ebuf.at[0]], rows_, gsem).wait()

        @functools.partial(plsc.parallel_loop(0, B // 16, 1, unroll=2))
        def scale_group(g):
            wv = lax.bitcast_convert_type(ebuf[1, pl.ds(g * 16, 16)],
                                          jnp.float32)
            for l in range(16):
                wbc = _bcast_lane(wv, l)
                e = g * 16 + l
                for q in range(QL):
                    sl = pl.ds(q * 16, 16)
                    rows_[e, sl] = rows_[e, sl] * wbc
        h = pltpu.async_copy(rows_, acc.at[dstall.at[b]], ssem, add=True)
        if fire_next:
            fire_eload(p, b + NSET)
        return h

    def layer_body(_, carry):
        fire_eload(0, 0)
        fire_eload(1, 1)
        load_and_fire(0, 0, drain=False)
        load_and_fire(1, 1, drain=False)
        finish(0, 0)
        load_and_fire(0, 2, drain=True)
        finish(1, 1)

        def ring_body(j, cr):
            load_and_fire(1, 2 * j + 1, drain=True)
            finish(0, 2 * j)
            load_and_fire(0, 2 * j + 2, drain=True)
            finish(1, 2 * j + 1)
            return cr
        lax.fori_loop(1, NBATCH // 2 - 1, ring_body, 0)
        load_and_fire(1, NBATCH - 1, drain=True)
        finish(0, NBATCH - 2, fire_next=False)
        finish(1, NBATCH - 1, fire_next=False)
        for p in range(NSET):
            ebuf, rows_ = sets[p]
            pltpu.make_async_copy(rows_, acc.at[dstall.at[NBATCH - NSET + p]],
                                  ssem).wait()
        plsc.subcore_barrier()

        for k in range(NCH):
            pltpu.sync_copy(x0.at[pl.ds(myrow + k * CH, CH)], cb0)
            pltpu.sync_copy(acc.at[pl.ds(accrow + k * CH, CH)], cb1)

            @functools.partial(plsc.parallel_loop(0, CH, 1, unroll=4))
            def addrow(r):
                for q in range(QL):
                    sl = pl.ds(q * 16, 16)
                    cb0[r, sl] = cb0[r, sl] + cb1[r, sl]
            pltpu.sync_copy(cb0, out.at[pl.ds(myrow + k * CH, CH)])
            pltpu.sync_copy(zbuf, acc.at[pl.ds(accrow + k * CH, CH)])
        plsc.subcore_barrier()
        return carry

    lax.fori_loop(0, NLAYER, layer_body, 0)


def kernel(edge_index, edge_weight, uEmbeds, iEmbeds):
    emb = jnp.concatenate([uEmbeds, iEmbeds], axis=0)
    pad = jnp.zeros((NP - N, HALF), jnp.float32)
    x0 = jnp.concatenate([emb[:, :HALF], pad, emb[:, HALF:], pad], axis=0)
    esw = jnp.stack([edge_index[1],
                     lax.bitcast_convert_type(edge_weight, jnp.int32)])
    edst4 = edge_index[0].reshape(NS, NBATCH, B)
    out2 = _lightgcn(x0, esw, edst4)
    out = jnp.concatenate([out2[:N], out2[NP:NP + N]], axis=1)
    return (out[:USER], out[USER:])
